# baseline (speedup 1.0000x reference)
import functools

import jax
import jax.numpy as jnp
from jax import lax
from jax.experimental import pallas as pl
from jax.experimental.pallas import tpu as pltpu
from jax.experimental.pallas import tpu_sc as plsc

L = 4
H = 128
NG = 64
N = 10000
E = 320000
DF = 128
DE = 16

NP = 10240
BN = 1024
NB = NP // BN
BE = 1024
CH = 88
NSLOT = 116
EP = 32 * NSLOT * CH
NBE = EP // 1024
RPT = NP // 16

_DOT = dict(preferred_element_type=jnp.float32, precision=lax.Precision.HIGHEST)


@functools.cache
def _build_sc_msg():
    mesh = plsc.VectorSubcoreMesh(core_axis_name="c", subcore_axis_name="s")

    @functools.partial(
        pl.kernel, mesh=mesh,
        out_type=jax.ShapeDtypeStruct((2, NP, H), jnp.float32),
        scratch_types=[
            pltpu.VMEM((CH,), jnp.int32),
            pltpu.VMEM((CH,), jnp.int32),
            pltpu.VMEM((CH,), jnp.int32),
            pltpu.VMEM((CH,), jnp.int32),
            pltpu.VMEM((CH, H), jnp.float32),
            pltpu.VMEM((CH, H), jnp.float32),
            pltpu.VMEM((CH, H), jnp.float32),
            pltpu.VMEM((CH, H), jnp.float32),
            pltpu.VMEM_SHARED((NP, H), jnp.float32),
            pltpu.SemaphoreType.DMA,
            pltpu.SemaphoreType.DMA,
            pltpu.SemaphoreType.DMA,
            pltpu.SemaphoreType.DMA,
            pltpu.SemaphoreType.DMA,
            pltpu.SemaphoreType.DMA,
            pltpu.SemaphoreType.DMA,
            pltpu.SemaphoreType.DMA,
        ],
    )
    def sc_msg(h_hbm, e_hbm, src_hbm, dst_hbm, out_hbm,
               srcv0, srcv1, dstv0, dstv1, rows0, rows1, ev0, ev1,
               aggr, si0, si1, sg0, sg1, se0, se1, ss0, ss1):
        c = lax.axis_index("c")
        s = lax.axis_index("s")
        srcv = {0: srcv0, 1: srcv1}
        dstv = {0: dstv0, 1: dstv1}
        rows = {0: rows0, 1: rows1}
        ev = {0: ev0, 1: ev1}
        semi = {0: si0, 1: si1}
        semg = {0: sg0, 1: sg1}
        seme = {0: se0, 1: se1}
        sems = {0: ss0, 1: ss1}

        def zrow(r, carry):
            for jj in range(H // 16):
                rows0[r, pl.ds(jj * 16, 16)] = jnp.zeros((16,), jnp.float32)
            return carry
        lax.fori_loop(0, CH, zrow, 0)
        for jj in range(RPT // CH):
            pltpu.sync_copy(rows0, aggr.at[pl.ds(s * RPT + jj * CH, CH)])
        _rem = RPT % CH
        if _rem:
            pltpu.sync_copy(
                rows0.at[pl.ds(0, _rem)],
                aggr.at[pl.ds(s * RPT + (RPT // CH) * CH, _rem)])
        plsc.subcore_barrier()

        w = c * 16 + s

        def base(j):
            return (w + 32 * j) * CH

        def issue_idx(j, b):
            pltpu.make_async_copy(src_hbm.at[pl.ds(base(j), CH)],
                                  srcv[b], semi[b]).start()
            pltpu.make_async_copy(dst_hbm.at[pl.ds(base(j), CH)],
                                  dstv[b], semi[b]).start()

        def wait_idx(b):
            pltpu.make_async_copy(src_hbm.at[pl.ds(0, CH)], srcv[b],
                                  semi[b]).wait()
            pltpu.make_async_copy(dst_hbm.at[pl.ds(0, CH)], dstv[b],
                                  semi[b]).wait()

        def issue_body(j, b):
            pltpu.make_async_copy(h_hbm.at[srcv[b]], rows[b],
                                  semg[b]).start()
            pltpu.make_async_copy(e_hbm.at[pl.ds(base(j), CH)],
                                  ev[b], seme[b]).start()

        def wait_body(b):
            pltpu.make_async_copy(h_hbm.at[srcv[b]], rows[b],
                                  semg[b]).wait()
            pltpu.make_async_copy(e_hbm.at[pl.ds(0, CH)], ev[b],
                                  seme[b]).wait()

        issue_idx(0, 0)
        wait_idx(0)
        issue_body(0, 0)
        issue_idx(1, 1)

        def scat_copy(b):
            return pltpu.make_async_copy(rows[b], aggr.at[dstv[b]], sems[b])

        def slot(j, b):
            nb = 1 - b

            @pl.when(j + 1 < NSLOT)
            def _():
                wait_idx(nb)

                @pl.when(j >= 1)
                def _():
                    scat_copy(nb).wait()
                issue_body(j + 1, nb)
            wait_body(b)

            def rowf(r, inner):
                rb = rows[b]
                eb = ev[b]
                for jj in range(H // 16):
                    sl = pl.ds(jj * 16, 16)
                    rb[r, sl] = jnp.maximum(rb[r, sl] + eb[r, sl], 0.0)
                return inner
            lax.fori_loop(0, CH, rowf, 0)
            pltpu.async_copy(rows[b], aggr.at[dstv[b]], sems[b], add=True)

            @pl.when(j + 2 < NSLOT)
            def _():
                issue_idx(j + 2, b)

        def pair(g, carry):
            slot(2 * g, 0)
            slot(2 * g + 1, 1)
            return carry
        lax.fori_loop(0, NSLOT // 2, pair, 0)
        scat_copy(0).wait()
        scat_copy(1).wait()

        plsc.subcore_barrier()
        pltpu.sync_copy(aggr.at[pl.ds(s * RPT, RPT)],
                        out_hbm.at[c, pl.ds(s * RPT, RPT)])

    return sc_msg


def _pre_kernel(x_ref, w_ref, b_ref, bmt_ref, h_ref, cnt_ref, acc):
    i = pl.program_id(0)

    @pl.when(i == 0)
    def _():
        acc[...] = jnp.zeros_like(acc)

    ridx = i * BN + lax.broadcasted_iota(jnp.int32, (BN, 1), 0)
    hv = jnp.dot(x_ref[...], w_ref[...], **_DOT) + b_ref[...]
    h_ref[...] = jnp.where(ridx < N, hv, 0.0)
    acc[...] += jnp.dot(bmt_ref[...], jnp.ones((BN, 1), jnp.float32), **_DOT)

    @pl.when(i == NB - 1)
    def _():
        cnt_ref[...] = acc[...]


def _k_pre(x, node_W, node_b, BmatT):
    return pl.pallas_call(
        _pre_kernel,
        grid=(NB,),
        in_specs=[
            pl.BlockSpec((BN, DF), lambda i: (i, 0)),
            pl.BlockSpec((DF, H), lambda i: (0, 0)),
            pl.BlockSpec((1, H), lambda i: (0, 0)),
            pl.BlockSpec((NG, BN), lambda i: (0, i)),
        ],
        out_specs=[
            pl.BlockSpec((BN, H), lambda i: (i, 0)),
            pl.BlockSpec((NG, 1), lambda i: (0, 0)),
        ],
        out_shape=[
            jax.ShapeDtypeStruct((NP, H), jnp.float32),
            jax.ShapeDtypeStruct((NG, 1), jnp.float32),
        ],
        scratch_shapes=[pltpu.VMEM((NG, 1), jnp.float32)],
    )(x, node_W, node_b, BmatT)


def _e_kernel(ea_ref, c_ref, d_ref, e_ref):
    e_ref[...] = jnp.dot(ea_ref[...], c_ref[...], **_DOT) + d_ref[...]


def _k_e(edge_attr, Ci, di):
    return pl.pallas_call(
        _e_kernel,
        grid=(NBE,),
        in_specs=[
            pl.BlockSpec((BE, DE), lambda i: (i, 0)),
            pl.BlockSpec((DE, H), lambda i: (0, 0)),
            pl.BlockSpec((1, H), lambda i: (0, 0)),
        ],
        out_specs=pl.BlockSpec((BE, H), lambda i: (i, 0)),
        out_shape=jax.ShapeDtypeStruct((EP, H), jnp.float32),
    )(edge_attr, Ci, di)


def _l1_kernel(h_ref, p_ref, heps_ref, w1_ref, b1_ref, w2_ref, b2_ref,
               bmt_ref, out_ref, s_ref, q_ref, g_ref, accS, accQ, accG):
    i = pl.program_id(0)

    @pl.when(i == 0)
    def _():
        accS[...] = jnp.zeros_like(accS)
        accQ[...] = jnp.zeros_like(accQ)
        accG[...] = jnp.zeros_like(accG)

    out0 = heps_ref[...] * h_ref[...] + p_ref[0] + p_ref[1]
    t = jnp.maximum(jnp.dot(out0, w1_ref[...], **_DOT) + b1_ref[...], 0.0)
    o = jnp.dot(t, w2_ref[...], **_DOT) + b2_ref[...]
    out_ref[...] = o
    ridx = i * BN + lax.broadcasted_iota(jnp.int32, (BN, 1), 0)
    om = jnp.where(ridx < N, o, 0.0)
    accS[...] += jnp.sum(om, axis=0, keepdims=True)
    accQ[...] += jnp.sum(om * om, axis=0, keepdims=True)
    accG[...] += jnp.dot(bmt_ref[...], o, **_DOT)

    @pl.when(i == NB - 1)
    def _():
        s_ref[...] = accS[...]
        q_ref[...] = accQ[...]
        g_ref[...] = accG[...]


def _k_node1(h, parts, heps, W1i, b1i, W2i, b2i, BmatT):
    return pl.pallas_call(
        _l1_kernel,
        grid=(NB,),
        in_specs=[
            pl.BlockSpec((BN, H), lambda i: (i, 0)),
            pl.BlockSpec((2, BN, H), lambda i: (0, i, 0)),
            pl.BlockSpec((1, H), lambda i: (0, 0)),
            pl.BlockSpec((H, H), lambda i: (0, 0)),
            pl.BlockSpec((1, H), lambda i: (0, 0)),
            pl.BlockSpec((H, H), lambda i: (0, 0)),
            pl.BlockSpec((1, H), lambda i: (0, 0)),
            pl.BlockSpec((NG, BN), lambda i: (0, i)),
        ],
        out_specs=[
            pl.BlockSpec((BN, H), lambda i: (i, 0)),
            pl.BlockSpec((1, H), lambda i: (0, 0)),
            pl.BlockSpec((1, H), lambda i: (0, 0)),
            pl.BlockSpec((NG, H), lambda i: (0, 0)),
        ],
        out_shape=[
            jax.ShapeDtypeStruct((NP, H), jnp.float32),
            jax.ShapeDtypeStruct((1, H), jnp.float32),
            jax.ShapeDtypeStruct((1, H), jnp.float32),
            jax.ShapeDtypeStruct((NG, H), jnp.float32),
        ],
        scratch_shapes=[
            pltpu.VMEM((1, H), jnp.float32),
            pltpu.VMEM((1, H), jnp.float32),
            pltpu.VMEM((NG, H), jnp.float32),
        ],
    )(h, parts, heps, W1i, b1i, W2i, b2i, BmatT)


def _l2_kernel(out_ref, s_ref, q_ref, g_ref, cnt_ref, bm_ref, bmt_ref,
               gam_ref, bet_ref, u_ref, ssq_ref, accUU):
    i = pl.program_id(0)

    @pl.when(i == 0)
    def _():
        accUU[...] = jnp.zeros_like(accUU)

    mu = s_ref[...] * (1.0 / N)
    var = q_ref[...] * (1.0 / N) - mu * mu
    scale = lax.rsqrt(var + 1e-5) * gam_ref[...]
    shift = bet_ref[...] - mu * scale
    o = out_ref[...]
    bn = o * scale + shift
    cnt = cnt_ref[...]
    safe = jnp.maximum(cnt, 1.0)
    mean_g = (g_ref[...] * scale + cnt * shift) / safe
    u = bn - jnp.dot(bm_ref[...], mean_g, **_DOT)
    u_ref[...] = u
    accUU[...] += jnp.dot(bmt_ref[...], u * u, **_DOT)

    @pl.when(i == NB - 1)
    def _():
        ssq_ref[...] = jnp.sum(accUU[...], axis=1, keepdims=True)


def _k_node2(out, S, Q, G, cnt, Bmat, BmatT, gam, bet):
    return pl.pallas_call(
        _l2_kernel,
        grid=(NB,),
        in_specs=[
            pl.BlockSpec((BN, H), lambda i: (i, 0)),
            pl.BlockSpec((1, H), lambda i: (0, 0)),
            pl.BlockSpec((1, H), lambda i: (0, 0)),
            pl.BlockSpec((NG, H), lambda i: (0, 0)),
            pl.BlockSpec((NG, 1), lambda i: (0, 0)),
            pl.BlockSpec((BN, NG), lambda i: (i, 0)),
            pl.BlockSpec((NG, BN), lambda i: (0, i)),
            pl.BlockSpec((1, H), lambda i: (0, 0)),
            pl.BlockSpec((1, H), lambda i: (0, 0)),
        ],
        out_specs=[
            pl.BlockSpec((BN, H), lambda i: (i, 0)),
            pl.BlockSpec((NG, 1), lambda i: (0, 0)),
        ],
        out_shape=[
            jax.ShapeDtypeStruct((NP, H), jnp.float32),
            jax.ShapeDtypeStruct((NG, 1), jnp.float32),
        ],
        scratch_shapes=[pltpu.VMEM((NG, H), jnp.float32)],
    )(out, S, Q, G, cnt, Bmat, BmatT, gam, bet)


def _l3_kernel(u_ref, ssq_ref, cnt_ref, bm_ref, h_ref):
    safe = jnp.maximum(cnt_ref[...], 1.0)
    scale_g = lax.rsqrt(1e-5 + ssq_ref[...] / safe)
    rs = jnp.dot(bm_ref[...], scale_g, **_DOT)
    h_ref[...] = jnp.maximum(u_ref[...] * rs, 0.0)


def _k_node3(u, SSQ, cnt, Bmat):
    return pl.pallas_call(
        _l3_kernel,
        grid=(NB,),
        in_specs=[
            pl.BlockSpec((BN, H), lambda i: (i, 0)),
            pl.BlockSpec((NG, 1), lambda i: (0, 0)),
            pl.BlockSpec((NG, 1), lambda i: (0, 0)),
            pl.BlockSpec((BN, NG), lambda i: (i, 0)),
        ],
        out_specs=pl.BlockSpec((BN, H), lambda i: (i, 0)),
        out_shape=jax.ShapeDtypeStruct((NP, H), jnp.float32),
    )(u, SSQ, cnt, Bmat)


def _ro_kernel(h_ref, bm_ref, bmt_ref, cnt_ref, aw1_ref, ab1_ref, aw2_ref,
               ab2_ref, out_ref, accG, accP, accD, accM):
    i = pl.program_id(0)

    @pl.when(i == 0)
    def _():
        accG[...] = jnp.zeros_like(accG)
        accP[...] = jnp.zeros_like(accP)
        accD[...] = jnp.zeros_like(accD)
        accM[...] = jnp.zeros_like(accM)

    h = h_ref[...]
    bm = bm_ref[...]
    accG[...] += jnp.dot(bmt_ref[...], h, **_DOT)
    a = jnp.tanh(jnp.dot(h, aw1_ref[...], **_DOT) + ab1_ref[...])
    att = jnp.dot(a, aw2_ref[...], **_DOT) + ab2_ref[...]
    w = jnp.exp(att)
    accD[...] += jnp.dot(bmt_ref[...], w, **_DOT)
    accP[...] += jnp.dot(bmt_ref[...], h * w, **_DOT)
    for g in range(NG):
        pres = jnp.sum(bm[:, g:g + 1]) > 0.0

        @pl.when(pres)
        def _(g=g):
            colmax = jnp.max(h + (bm[:, g:g + 1] - 1.0) * 1e30,
                             axis=0, keepdims=True)
            accM[pl.ds(g, 1), :] = jnp.maximum(accM[pl.ds(g, 1), :], colmax)

    @pl.when(i == NB - 1)
    def _():
        safe = jnp.maximum(cnt_ref[...], 1.0)
        out_ref[:, :H] = accG[...] / safe
        out_ref[:, H:2 * H] = jnp.maximum(accM[...], 0.0)
        out_ref[:, 2 * H:] = accP[...] / (accD[...] + 1e-8)


def _k_readout(h, Bmat, BmatT, cnt, aW1, ab1, aW2, ab2):
    return pl.pallas_call(
        _ro_kernel,
        grid=(NB,),
        in_specs=[
            pl.BlockSpec((BN, H), lambda i: (i, 0)),
            pl.BlockSpec((BN, NG), lambda i: (i, 0)),
            pl.BlockSpec((NG, BN), lambda i: (0, i)),
            pl.BlockSpec((NG, 1), lambda i: (0, 0)),
            pl.BlockSpec((H, H // 2), lambda i: (0, 0)),
            pl.BlockSpec((1, H // 2), lambda i: (0, 0)),
            pl.BlockSpec((H // 2, 1), lambda i: (0, 0)),
            pl.BlockSpec((1, 1), lambda i: (0, 0)),
        ],
        out_specs=pl.BlockSpec((NG, 3 * H), lambda i: (0, 0)),
        out_shape=jax.ShapeDtypeStruct((NG, 3 * H), jnp.float32),
        scratch_shapes=[
            pltpu.VMEM((NG, H), jnp.float32),
            pltpu.VMEM((NG, H), jnp.float32),
            pltpu.VMEM((NG, 1), jnp.float32),
            pltpu.VMEM((NG, H), jnp.float32),
        ],
    )(h, Bmat, BmatT, cnt, aW1, ab1, aW2, ab2)


def kernel(x, edge_index, edge_attr, batch, node_W, node_b, edge_W, edge_b,
           eps, Wedge, bedge, W1, b1, W2, b2, gamma, beta, aW1, ab1, aW2, ab2):
    src = jnp.zeros((EP,), jnp.int32).at[:E].set(edge_index[0])
    dst = jnp.full((EP,), N, jnp.int32).at[:E].set(edge_index[1])
    eap = jnp.zeros((EP, DE), jnp.float32).at[:E].set(edge_attr)
    xp = jnp.zeros((NP, DF), jnp.float32).at[:N].set(x)
    batchp = jnp.full((NP,), NG, dtype=batch.dtype).at[:N].set(batch)
    Bmat = (batchp[:, None] == jnp.arange(NG, dtype=batch.dtype)[None, :])
    Bmat = Bmat.astype(jnp.float32)
    BmatT = Bmat.T
    Ci = jnp.einsum('dh,lhk->ldk', edge_W, Wedge)
    di = jnp.einsum('h,lhk->lk', edge_b, Wedge) + bedge


    h, cnt = _k_pre(xp, node_W, node_b.reshape(1, H), BmatT)
    for i in range(L):
        e = _k_e(eap, Ci[i], di[i].reshape(1, H))
        parts = _build_sc_msg()(h, e, src, dst)
        heps = (1.0 + eps[i]) * jnp.ones((1, H), jnp.float32)
        out, S, Q, G = _k_node1(h, parts, heps, W1[i], b1[i].reshape(1, H),
                                W2[i], b2[i].reshape(1, H), BmatT)
        u, SSQ = _k_node2(out, S, Q, G, cnt, Bmat, BmatT,
                          gamma[i].reshape(1, H), beta[i].reshape(1, H))
        h = _k_node3(u, SSQ, cnt, Bmat)

    return _k_readout(h, Bmat, BmatT, cnt, aW1, ab1.reshape(1, H // 2),
                      aW2, ab2.reshape(1, 1))


# speedup vs baseline: 1.1785x; 1.1785x over previous
import functools

import jax
import jax.numpy as jnp
from jax import lax
from jax.experimental import pallas as pl
from jax.experimental.pallas import tpu as pltpu
from jax.experimental.pallas import tpu_sc as plsc

L = 4
H = 128
NG = 64
N = 10000
E = 320000
DF = 128
DE = 16

NP = 10240
BN = 1024
NB = NP // BN
BE = 1024
CH = 64
NSLOT = 158
EP = 32 * NSLOT * CH
NBE = EP // 1024
RPT = NP // 16

_DOT = dict(preferred_element_type=jnp.float32, precision=lax.Precision.HIGHEST)


@functools.cache
def _build_sc_msg():
    mesh = plsc.VectorSubcoreMesh(core_axis_name="c", subcore_axis_name="s")

    @functools.partial(
        pl.kernel, mesh=mesh,
        out_type=jax.ShapeDtypeStruct((2, NP, H), jnp.float32),
        scratch_types=[
            pltpu.VMEM((CH,), jnp.int32),
            pltpu.VMEM((CH,), jnp.int32),
            pltpu.VMEM((CH,), jnp.int32),
            pltpu.VMEM((CH,), jnp.int32),
            pltpu.VMEM((CH, H), jnp.float32),
            pltpu.VMEM((CH, H), jnp.float32),
            pltpu.VMEM((CH, H), jnp.float32),
            pltpu.VMEM((CH, H), jnp.float32),
            pltpu.VMEM_SHARED((NP, H), jnp.float32),
            pltpu.SemaphoreType.DMA,
            pltpu.SemaphoreType.DMA,
            pltpu.SemaphoreType.DMA,
            pltpu.SemaphoreType.DMA,
            pltpu.SemaphoreType.DMA,
            pltpu.SemaphoreType.DMA,
            pltpu.SemaphoreType.DMA,
            pltpu.SemaphoreType.DMA,
        ],
    )
    def sc_msg(h_hbm, e_hbm, src_hbm, dst_hbm, out_hbm,
               srcv0, srcv1, dstv0, dstv1, rows0, rows1, ev0, ev1,
               aggr, si0, si1, sg0, sg1, se0, se1, ss0, ss1):
        c = lax.axis_index("c")
        s = lax.axis_index("s")
        srcv = {0: srcv0, 1: srcv1}
        dstv = {0: dstv0, 1: dstv1}
        rows = {0: rows0, 1: rows1}
        ev = {0: ev0, 1: ev1}
        semi = {0: si0, 1: si1}
        semg = {0: sg0, 1: sg1}
        seme = {0: se0, 1: se1}
        sems = {0: ss0, 1: ss1}

        def zrow(r, carry):
            for jj in range(H // 16):
                rows0[r, pl.ds(jj * 16, 16)] = jnp.zeros((16,), jnp.float32)
            return carry
        lax.fori_loop(0, CH, zrow, 0)
        for jj in range(RPT // CH):
            pltpu.sync_copy(rows0, aggr.at[pl.ds(s * RPT + jj * CH, CH)])
        _rem = RPT % CH
        if _rem:
            pltpu.sync_copy(
                rows0.at[pl.ds(0, _rem)],
                aggr.at[pl.ds(s * RPT + (RPT // CH) * CH, _rem)])
        plsc.subcore_barrier()

        w = c * 16 + s

        def base(j):
            return (w + 32 * j) * CH

        def issue_idx(j, b):
            pltpu.make_async_copy(src_hbm.at[pl.ds(base(j), CH)],
                                  srcv[b], semi[b]).start()
            pltpu.make_async_copy(dst_hbm.at[pl.ds(base(j), CH)],
                                  dstv[b], semi[b]).start()

        def wait_idx(b):
            pltpu.make_async_copy(src_hbm.at[pl.ds(0, CH)], srcv[b],
                                  semi[b]).wait()
            pltpu.make_async_copy(dst_hbm.at[pl.ds(0, CH)], dstv[b],
                                  semi[b]).wait()

        def issue_body(j, b):
            pltpu.make_async_copy(h_hbm.at[srcv[b]], rows[b],
                                  semg[b]).start()
            pltpu.make_async_copy(e_hbm.at[pl.ds(base(j), CH)],
                                  ev[b], seme[b]).start()

        def wait_body(b):
            pltpu.make_async_copy(h_hbm.at[srcv[b]], rows[b],
                                  semg[b]).wait()
            pltpu.make_async_copy(e_hbm.at[pl.ds(0, CH)], ev[b],
                                  seme[b]).wait()

        issue_idx(0, 0)
        wait_idx(0)
        issue_body(0, 0)
        issue_idx(1, 1)

        def scat_copy(b):
            return pltpu.make_async_copy(rows[b], aggr.at[dstv[b]], sems[b])

        def slot(j, b):
            nb = 1 - b

            @pl.when(j + 1 < NSLOT)
            def _():
                wait_idx(nb)

                @pl.when(j >= 1)
                def _():
                    scat_copy(nb).wait()
                issue_body(j + 1, nb)
            wait_body(b)

            def rowf(r, inner):
                rb = rows[b]
                eb = ev[b]
                for jj in range(H // 16):
                    sl = pl.ds(jj * 16, 16)
                    rb[r, sl] = jnp.maximum(rb[r, sl] + eb[r, sl], 0.0)
                return inner
            lax.fori_loop(0, CH, rowf, 0)
            pltpu.async_copy(rows[b], aggr.at[dstv[b]], sems[b], add=True)

            @pl.when(j + 2 < NSLOT)
            def _():
                issue_idx(j + 2, b)

        def pair(g, carry):
            slot(2 * g, 0)
            slot(2 * g + 1, 1)
            return carry
        lax.fori_loop(0, NSLOT // 2, pair, 0)
        scat_copy(0).wait()
        scat_copy(1).wait()

        plsc.subcore_barrier()
        pltpu.sync_copy(aggr.at[pl.ds(s * RPT, RPT)],
                        out_hbm.at[c, pl.ds(s * RPT, RPT)])

    return sc_msg


def _pre_kernel(x_ref, w_ref, b_ref, bmt_ref, h_ref, cnt_ref, acc):
    i = pl.program_id(0)

    @pl.when(i == 0)
    def _():
        acc[...] = jnp.zeros_like(acc)

    ridx = i * BN + lax.broadcasted_iota(jnp.int32, (BN, 1), 0)
    hv = jnp.dot(x_ref[...], w_ref[...], **_DOT) + b_ref[...]
    h_ref[...] = jnp.where(ridx < N, hv, 0.0)
    acc[...] += jnp.dot(bmt_ref[...], jnp.ones((BN, 1), jnp.float32), **_DOT)

    @pl.when(i == NB - 1)
    def _():
        cnt_ref[...] = acc[...]


def _k_pre(x, node_W, node_b, BmatT):
    return pl.pallas_call(
        _pre_kernel,
        grid=(NB,),
        in_specs=[
            pl.BlockSpec((BN, DF), lambda i: (i, 0)),
            pl.BlockSpec((DF, H), lambda i: (0, 0)),
            pl.BlockSpec((1, H), lambda i: (0, 0)),
            pl.BlockSpec((NG, BN), lambda i: (0, i)),
        ],
        out_specs=[
            pl.BlockSpec((BN, H), lambda i: (i, 0)),
            pl.BlockSpec((NG, 1), lambda i: (0, 0)),
        ],
        out_shape=[
            jax.ShapeDtypeStruct((NP, H), jnp.float32),
            jax.ShapeDtypeStruct((NG, 1), jnp.float32),
        ],
        scratch_shapes=[pltpu.VMEM((NG, 1), jnp.float32)],
    )(x, node_W, node_b, BmatT)


def _e_kernel(ea_ref, c_ref, d_ref, e_ref):
    e_ref[...] = jnp.dot(ea_ref[...], c_ref[...], **_DOT) + d_ref[...]


def _k_e(edge_attr, Ci, di):
    return pl.pallas_call(
        _e_kernel,
        grid=(NBE,),
        in_specs=[
            pl.BlockSpec((BE, DE), lambda i: (i, 0)),
            pl.BlockSpec((DE, H), lambda i: (0, 0)),
            pl.BlockSpec((1, H), lambda i: (0, 0)),
        ],
        out_specs=pl.BlockSpec((BE, H), lambda i: (i, 0)),
        out_shape=jax.ShapeDtypeStruct((EP, H), jnp.float32),
    )(edge_attr, Ci, di)


def _l1_kernel(h_ref, p_ref, heps_ref, w1_ref, b1_ref, w2_ref, b2_ref,
               bmt_ref, out_ref, s_ref, q_ref, g_ref, accS, accQ, accG):
    i = pl.program_id(0)

    @pl.when(i == 0)
    def _():
        accS[...] = jnp.zeros_like(accS)
        accQ[...] = jnp.zeros_like(accQ)
        accG[...] = jnp.zeros_like(accG)

    out0 = heps_ref[...] * h_ref[...] + p_ref[0] + p_ref[1]
    t = jnp.maximum(jnp.dot(out0, w1_ref[...], **_DOT) + b1_ref[...], 0.0)
    o = jnp.dot(t, w2_ref[...], **_DOT) + b2_ref[...]
    out_ref[...] = o
    ridx = i * BN + lax.broadcasted_iota(jnp.int32, (BN, 1), 0)
    om = jnp.where(ridx < N, o, 0.0)
    accS[...] += jnp.sum(om, axis=0, keepdims=True)
    accQ[...] += jnp.sum(om * om, axis=0, keepdims=True)
    accG[...] += jnp.dot(bmt_ref[...], o, **_DOT)

    @pl.when(i == NB - 1)
    def _():
        s_ref[...] = accS[...]
        q_ref[...] = accQ[...]
        g_ref[...] = accG[...]


def _k_node1(h, parts, heps, W1i, b1i, W2i, b2i, BmatT):
    return pl.pallas_call(
        _l1_kernel,
        grid=(NB,),
        in_specs=[
            pl.BlockSpec((BN, H), lambda i: (i, 0)),
            pl.BlockSpec((2, BN, H), lambda i: (0, i, 0)),
            pl.BlockSpec((1, H), lambda i: (0, 0)),
            pl.BlockSpec((H, H), lambda i: (0, 0)),
            pl.BlockSpec((1, H), lambda i: (0, 0)),
            pl.BlockSpec((H, H), lambda i: (0, 0)),
            pl.BlockSpec((1, H), lambda i: (0, 0)),
            pl.BlockSpec((NG, BN), lambda i: (0, i)),
        ],
        out_specs=[
            pl.BlockSpec((BN, H), lambda i: (i, 0)),
            pl.BlockSpec((1, H), lambda i: (0, 0)),
            pl.BlockSpec((1, H), lambda i: (0, 0)),
            pl.BlockSpec((NG, H), lambda i: (0, 0)),
        ],
        out_shape=[
            jax.ShapeDtypeStruct((NP, H), jnp.float32),
            jax.ShapeDtypeStruct((1, H), jnp.float32),
            jax.ShapeDtypeStruct((1, H), jnp.float32),
            jax.ShapeDtypeStruct((NG, H), jnp.float32),
        ],
        scratch_shapes=[
            pltpu.VMEM((1, H), jnp.float32),
            pltpu.VMEM((1, H), jnp.float32),
            pltpu.VMEM((NG, H), jnp.float32),
        ],
    )(h, parts, heps, W1i, b1i, W2i, b2i, BmatT)


def _l2_kernel(out_ref, s_ref, q_ref, g_ref, cnt_ref, bm_ref, bmt_ref,
               gam_ref, bet_ref, u_ref, ssq_ref, accUU):
    i = pl.program_id(0)

    @pl.when(i == 0)
    def _():
        accUU[...] = jnp.zeros_like(accUU)

    mu = s_ref[...] * (1.0 / N)
    var = q_ref[...] * (1.0 / N) - mu * mu
    scale = lax.rsqrt(var + 1e-5) * gam_ref[...]
    shift = bet_ref[...] - mu * scale
    o = out_ref[...]
    bn = o * scale + shift
    cnt = cnt_ref[...]
    safe = jnp.maximum(cnt, 1.0)
    mean_g = (g_ref[...] * scale + cnt * shift) / safe
    u = bn - jnp.dot(bm_ref[...], mean_g, **_DOT)
    u_ref[...] = u
    accUU[...] += jnp.dot(bmt_ref[...], u * u, **_DOT)

    @pl.when(i == NB - 1)
    def _():
        ssq_ref[...] = jnp.sum(accUU[...], axis=1, keepdims=True)


def _k_node2(out, S, Q, G, cnt, Bmat, BmatT, gam, bet):
    return pl.pallas_call(
        _l2_kernel,
        grid=(NB,),
        in_specs=[
            pl.BlockSpec((BN, H), lambda i: (i, 0)),
            pl.BlockSpec((1, H), lambda i: (0, 0)),
            pl.BlockSpec((1, H), lambda i: (0, 0)),
            pl.BlockSpec((NG, H), lambda i: (0, 0)),
            pl.BlockSpec((NG, 1), lambda i: (0, 0)),
            pl.BlockSpec((BN, NG), lambda i: (i, 0)),
            pl.BlockSpec((NG, BN), lambda i: (0, i)),
            pl.BlockSpec((1, H), lambda i: (0, 0)),
            pl.BlockSpec((1, H), lambda i: (0, 0)),
        ],
        out_specs=[
            pl.BlockSpec((BN, H), lambda i: (i, 0)),
            pl.BlockSpec((NG, 1), lambda i: (0, 0)),
        ],
        out_shape=[
            jax.ShapeDtypeStruct((NP, H), jnp.float32),
            jax.ShapeDtypeStruct((NG, 1), jnp.float32),
        ],
        scratch_shapes=[pltpu.VMEM((NG, H), jnp.float32)],
    )(out, S, Q, G, cnt, Bmat, BmatT, gam, bet)


def _l3_kernel(u_ref, ssq_ref, cnt_ref, bm_ref, h_ref):
    safe = jnp.maximum(cnt_ref[...], 1.0)
    scale_g = lax.rsqrt(1e-5 + ssq_ref[...] / safe)
    rs = jnp.dot(bm_ref[...], scale_g, **_DOT)
    h_ref[...] = jnp.maximum(u_ref[...] * rs, 0.0)


def _k_node3(u, SSQ, cnt, Bmat):
    return pl.pallas_call(
        _l3_kernel,
        grid=(NB,),
        in_specs=[
            pl.BlockSpec((BN, H), lambda i: (i, 0)),
            pl.BlockSpec((NG, 1), lambda i: (0, 0)),
            pl.BlockSpec((NG, 1), lambda i: (0, 0)),
            pl.BlockSpec((BN, NG), lambda i: (i, 0)),
        ],
        out_specs=pl.BlockSpec((BN, H), lambda i: (i, 0)),
        out_shape=jax.ShapeDtypeStruct((NP, H), jnp.float32),
    )(u, SSQ, cnt, Bmat)


def _ro_kernel(h_ref, bm_ref, bmt_ref, cnt_ref, aw1_ref, ab1_ref, aw2_ref,
               ab2_ref, out_ref, accG, accP, accD, accM):
    i = pl.program_id(0)

    @pl.when(i == 0)
    def _():
        accG[...] = jnp.zeros_like(accG)
        accP[...] = jnp.zeros_like(accP)
        accD[...] = jnp.zeros_like(accD)
        accM[...] = jnp.zeros_like(accM)

    h = h_ref[...]
    bm = bm_ref[...]
    accG[...] += jnp.dot(bmt_ref[...], h, **_DOT)
    a = jnp.tanh(jnp.dot(h, aw1_ref[...], **_DOT) + ab1_ref[...])
    att = jnp.dot(a, aw2_ref[...], **_DOT) + ab2_ref[...]
    w = jnp.exp(att)
    accD[...] += jnp.dot(bmt_ref[...], w, **_DOT)
    accP[...] += jnp.dot(bmt_ref[...], h * w, **_DOT)
    for g in range(NG):
        pres = jnp.sum(bm[:, g:g + 1]) > 0.0

        @pl.when(pres)
        def _(g=g):
            colmax = jnp.max(h + (bm[:, g:g + 1] - 1.0) * 1e30,
                             axis=0, keepdims=True)
            accM[pl.ds(g, 1), :] = jnp.maximum(accM[pl.ds(g, 1), :], colmax)

    @pl.when(i == NB - 1)
    def _():
        safe = jnp.maximum(cnt_ref[...], 1.0)
        out_ref[:, :H] = accG[...] / safe
        out_ref[:, H:2 * H] = jnp.maximum(accM[...], 0.0)
        out_ref[:, 2 * H:] = accP[...] / (accD[...] + 1e-8)


def _k_readout(h, Bmat, BmatT, cnt, aW1, ab1, aW2, ab2):
    return pl.pallas_call(
        _ro_kernel,
        grid=(NB,),
        in_specs=[
            pl.BlockSpec((BN, H), lambda i: (i, 0)),
            pl.BlockSpec((BN, NG), lambda i: (i, 0)),
            pl.BlockSpec((NG, BN), lambda i: (0, i)),
            pl.BlockSpec((NG, 1), lambda i: (0, 0)),
            pl.BlockSpec((H, H // 2), lambda i: (0, 0)),
            pl.BlockSpec((1, H // 2), lambda i: (0, 0)),
            pl.BlockSpec((H // 2, 1), lambda i: (0, 0)),
            pl.BlockSpec((1, 1), lambda i: (0, 0)),
        ],
        out_specs=pl.BlockSpec((NG, 3 * H), lambda i: (0, 0)),
        out_shape=jax.ShapeDtypeStruct((NG, 3 * H), jnp.float32),
        scratch_shapes=[
            pltpu.VMEM((NG, H), jnp.float32),
            pltpu.VMEM((NG, H), jnp.float32),
            pltpu.VMEM((NG, 1), jnp.float32),
            pltpu.VMEM((NG, H), jnp.float32),
        ],
    )(h, Bmat, BmatT, cnt, aW1, ab1, aW2, ab2)


def kernel(x, edge_index, edge_attr, batch, node_W, node_b, edge_W, edge_b,
           eps, Wedge, bedge, W1, b1, W2, b2, gamma, beta, aW1, ab1, aW2, ab2):
    src = jnp.zeros((EP,), jnp.int32).at[:E].set(edge_index[0])
    dst = jnp.full((EP,), N, jnp.int32).at[:E].set(edge_index[1])
    eap = jnp.zeros((EP, DE), jnp.float32).at[:E].set(edge_attr)
    xp = jnp.zeros((NP, DF), jnp.float32).at[:N].set(x)
    batchp = jnp.full((NP,), NG, dtype=batch.dtype).at[:N].set(batch)
    Bmat = (batchp[:, None] == jnp.arange(NG, dtype=batch.dtype)[None, :])
    Bmat = Bmat.astype(jnp.float32)
    BmatT = Bmat.T
    Ci = jnp.einsum('dh,lhk->ldk', edge_W, Wedge)
    di = jnp.einsum('h,lhk->lk', edge_b, Wedge) + bedge


    h, cnt = _k_pre(xp, node_W, node_b.reshape(1, H), BmatT)
    for i in range(L):
        e = _k_e(eap, Ci[i], di[i].reshape(1, H))
        parts = _build_sc_msg()(h, e, src, dst)
        heps = (1.0 + eps[i]) * jnp.ones((1, H), jnp.float32)
        out, S, Q, G = _k_node1(h, parts, heps, W1[i], b1[i].reshape(1, H),
                                W2[i], b2[i].reshape(1, H), BmatT)
        u, SSQ = _k_node2(out, S, Q, G, cnt, Bmat, BmatT,
                          gamma[i].reshape(1, H), beta[i].reshape(1, H))
        h = _k_node3(u, SSQ, cnt, Bmat)

    return _k_readout(h, Bmat, BmatT, cnt, aW1, ab1.reshape(1, H // 2),
                      aW2, ab2.reshape(1, 1))
